# baseline (speedup 1.0000x reference)
import functools

import jax
import jax.numpy as jnp
from jax import lax
from jax.experimental import pallas as pl
from jax.experimental.pallas import tpu as pltpu
from jax.experimental.pallas import tpu_sc as plsc

NC = 2
NS = 16
NW = NC * NS

V = 1000000
BATCH = 4096
L = 50
D = 32


def _make_lookup():
    mesh = plsc.VectorSubcoreMesh(core_axis_name="c", subcore_axis_name="s")

    @functools.partial(
        pl.kernel,
        mesh=mesh,
        compiler_params=pltpu.CompilerParams(use_tc_tiling_on_sc=False),
        out_type=jax.ShapeDtypeStruct((L * D, BATCH), jnp.float32),
        scratch_types=[
            pltpu.VMEM((BATCH,), jnp.int32),
            pltpu.VMEM((BATCH,), jnp.float32),
            pltpu.SemaphoreType.DMA,
        ],
    )
    def lookup(table_hbm, idx_hbm, out_hbm, idx_v, row_v, sem):
        d = lax.axis_index("s") * NC + lax.axis_index("c")

        def pos_body(l, carry):
            pltpu.sync_copy(idx_hbm.at[l], idx_v)
            pltpu.async_copy(table_hbm.at[d].at[idx_v], row_v, sem).wait()
            pltpu.sync_copy(row_v, out_hbm.at[l * D + d])
            return carry

        lax.fori_loop(0, L, pos_body, 0)

    return lookup


_lookup = _make_lookup()


@jax.jit
def kernel(inputs, embeddings):
    out2 = _lookup(embeddings.T, inputs.T.astype(jnp.int32))
    return out2.reshape(L, D, BATCH).transpose(2, 0, 1)


# speedup vs baseline: 4.6416x; 4.6416x over previous
import functools

import jax
import jax.numpy as jnp
from jax import lax
from jax.experimental import pallas as pl
from jax.experimental.pallas import tpu as pltpu
from jax.experimental.pallas import tpu_sc as plsc

NC = 2
NS = 16
NW = NC * NS

V = 1000000
BATCH = 4096
L = 50
D = 32
COLS = BATCH // NW
L_C = 10
N_CHUNK = L // L_C


def _make_lookup():
    mesh = plsc.VectorSubcoreMesh(core_axis_name="c", subcore_axis_name="s")

    @functools.partial(
        pl.kernel,
        mesh=mesh,
        compiler_params=pltpu.CompilerParams(use_tc_tiling_on_sc=False),
        out_type=jax.ShapeDtypeStruct((L, BATCH, D), jnp.float32),
        scratch_types=[
            pltpu.VMEM((L_C, COLS), jnp.int32),
            pltpu.VMEM((L_C, COLS, D), jnp.float32),
            pltpu.SemaphoreType.DMA,
            pltpu.SemaphoreType.DMA,
        ],
    )
    def lookup(table_hbm, idx_hbm, out_hbm, idx_v, rows_v, sem, sem2):
        w = lax.axis_index("s") * NC + lax.axis_index("c")
        col0 = w * COLS

        def chunk_body(c, carry):
            l0 = c * L_C
            pltpu.sync_copy(
                idx_hbm.at[pl.ds(l0, L_C), pl.ds(col0, COLS)], idx_v
            )
            gathers = [
                pltpu.async_copy(
                    table_hbm.at[idx_v.at[li]], rows_v.at[li], sem
                )
                for li in range(L_C)
            ]
            for h in gathers:
                h.wait()
            pltpu.sync_copy(
                rows_v, out_hbm.at[pl.ds(l0, L_C), pl.ds(col0, COLS)]
            )
            return carry

        lax.fori_loop(0, N_CHUNK, chunk_body, 0)

    return lookup


_lookup = _make_lookup()


@jax.jit
def kernel(inputs, embeddings):
    tflat = lax.optimization_barrier(embeddings.reshape(-1))
    out3 = _lookup(tflat.reshape(V, D), inputs.T.astype(jnp.int32))
    return out3.transpose(1, 0, 2)
